# baseline (speedup 1.0000x reference)
import functools

import jax
import jax.numpy as jnp
from jax import lax
from jax.experimental import pallas as pl
from jax.experimental.pallas import tpu as pltpu
from jax.experimental.pallas import tpu_sc as plsc

_NC = 2
_NS = 16
_NW = _NC * _NS
_L = 16

_W = 128
_C = 128
_NSET = 4
_LAG = _NSET - 1

_VB = 256


def _mesh():
    return plsc.VectorSubcoreMesh(
        core_axis_name="c", subcore_axis_name="s", num_cores=_NC, num_subcores=_NS
    )


@functools.lru_cache(maxsize=None)
def _make_xpose(v, d):
    n_full = v // _VB
    tail = v - n_full * _VB
    per_w = (n_full + _NW - 1) // _NW

    @functools.partial(
        pl.kernel,
        out_type=jax.ShapeDtypeStruct((v, _W), jnp.float32),
        mesh=_mesh(),
        compiler_params=pltpu.CompilerParams(needs_layout_passes=False),
        scratch_types=[
            pltpu.VMEM((2, _VB, _W), jnp.float32),
            [pltpu.SemaphoreType.DMA] * 2,
            [pltpu.SemaphoreType.DMA] * 2,
        ],
    )
    def xpose_kernel(tt_hbm, out_hbm, out_v, rsems, wsems):
        wid = lax.axis_index("s") * _NC + lax.axis_index("c")
        m_w = (n_full - wid + _NW - 1) // _NW

        def slab_of(k):
            return wid + k * _NW

        def issue_reads(k, b):
            for dd in range(d):
                pltpu.async_copy(
                    tt_hbm.at[dd, pl.ds(slab_of(k) * _VB, _VB)],
                    out_v.at[b, pl.ds(0, _VB), dd],
                    rsems[b],
                )

        def wait_reads(b):
            for dd in range(d):
                pltpu.make_async_copy(
                    tt_hbm.at[0, pl.ds(0, _VB)],
                    out_v.at[b, pl.ds(0, _VB), 0],
                    rsems[b],
                ).wait()

        def issue_write(k, b):
            pltpu.async_copy(
                out_v.at[b], out_hbm.at[pl.ds(slab_of(k) * _VB, _VB)], wsems[b]
            )

        def wait_write(b):
            pltpu.make_async_copy(
                out_v.at[b], out_hbm.at[pl.ds(0, _VB)], wsems[b]
            ).wait()

        def pair(j, carry):
            for u in range(2):
                k = j * 2 + u
                b = u

                @pl.when(k < m_w)
                def _do():
                    @pl.when(k >= 2)
                    def _ww():
                        wait_write(b)

                    issue_reads(k, b)
                    wait_reads(b)
                    issue_write(k, b)

            return carry

        lax.fori_loop(0, (per_w + 1) // 2, pair, 0)

        @pl.when(m_w >= 1)
        def _drain0():
            wait_write(0)

        @pl.when(m_w >= 2)
        def _drain1():
            wait_write(1)

    return xpose_kernel


@functools.lru_cache(maxsize=None)
def _make_gather(n, v, d):
    per_w = n // _NW
    n_sg = per_w // _C
    assert per_w * _NW == n and n_sg * _C == per_w
    assert n_sg % _NSET == 0 and n_sg > _NSET

    @functools.partial(
        pl.kernel,
        out_type=jax.ShapeDtypeStruct((n, _W), jnp.float32),
        mesh=_mesh(),
        compiler_params=pltpu.CompilerParams(use_tc_tiling_on_sc=False),
        scratch_types=[
            pltpu.VMEM((n_sg, _C), jnp.int32),
            pltpu.VMEM((_NSET, _C, _W), jnp.float32),
            [pltpu.SemaphoreType.DMA] * _NSET,
            [pltpu.SemaphoreType.DMA] * _NSET,
        ],
    )
    def gather_kernel(table_hbm, idx_hbm, out_hbm, idx_v, rows_v, gsems, wsems):
        wid = lax.axis_index("s") * _NC + lax.axis_index("c")
        base = wid * per_w
        pltpu.sync_copy(idx_hbm.at[wid], idx_v)

        def issue_gather(g, s):
            pltpu.async_copy(table_hbm.at[idx_v.at[g]], rows_v.at[s], gsems[s])

        def wait_gather(s):
            pltpu.make_async_copy(
                table_hbm.at[idx_v.at[0]], rows_v.at[s], gsems[s]
            ).wait()

        def issue_write(g, s):
            pltpu.async_copy(
                rows_v.at[s, pl.ds(0, _C), pl.ds(0, d)],
                out_hbm.at[pl.ds(base + g * _C, _C), pl.ds(0, d)],
                wsems[s],
            )

        def wait_write(s):
            pltpu.make_async_copy(
                rows_v.at[s, pl.ds(0, _C), pl.ds(0, d)],
                out_hbm.at[pl.ds(base, _C), pl.ds(0, d)],
                wsems[s],
            ).wait()

        for s in range(_LAG):
            issue_gather(s, s)

        def group(j, carry):
            for u in range(_NSET):
                sg = j * _NSET + u
                wait_gather(u)
                issue_write(sg, u)
                t = sg + _LAG
                s_t = (u + _LAG) % _NSET
                if u == 0:
                    @pl.when((t < n_sg) & (sg >= 1))
                    def _ww():
                        wait_write(s_t)
                else:
                    @pl.when(t < n_sg)
                    def _ww():
                        wait_write(s_t)

                @pl.when(t < n_sg)
                def _ig():
                    issue_gather(t, s_t)

            return carry

        lax.fori_loop(0, n_sg // _NSET, group, 0)
        for u in range(_NSET):
            wait_write(u)

    return gather_kernel


def kernel(x, table):
    b, l = x.shape
    v, d = table.shape
    n = b * l
    per_w = n // _NW
    table_p = _make_xpose(v, d)(table.T)
    n_full = (v // _VB) * _VB
    if n_full < v:
        tail_rows = jnp.pad(table[n_full:, :], ((0, 0), (0, _W - d)))
        table_p = lax.dynamic_update_slice(table_p, tail_rows, (n_full, 0))
    idx = x.reshape(_NW, per_w // _C, _C)
    out_p = _make_gather(n, v, d)(table_p, idx)
    return out_p[:, :d].reshape(b, l, d)


# speedup vs baseline: 7.7898x; 7.7898x over previous
import functools

import jax
import jax.numpy as jnp
from jax import lax
from jax.experimental import pallas as pl
from jax.experimental.pallas import tpu as pltpu
from jax.experimental.pallas import tpu_sc as plsc

_NC = 2
_NS = 16
_NW = _NC * _NS

_W = 128
_C = 128
_NSET = 4
_LAG = _NSET - 1


@functools.lru_cache(maxsize=None)
def _make_gather(n, v, d):
    per_w = n // _NW
    n_sg = per_w // _C
    assert per_w * _NW == n and n_sg * _C == per_w
    assert n_sg % _NSET == 0 and n_sg > _NSET

    mesh = plsc.VectorSubcoreMesh(
        core_axis_name="c", subcore_axis_name="s", num_cores=_NC, num_subcores=_NS
    )

    @functools.partial(
        pl.kernel,
        out_type=jax.ShapeDtypeStruct((n, _W), jnp.float32),
        mesh=mesh,
        compiler_params=pltpu.CompilerParams(use_tc_tiling_on_sc=False),
        scratch_types=[
            pltpu.VMEM((n_sg, _C), jnp.int32),
            pltpu.VMEM((_NSET, _C, _W), jnp.float32),
            [pltpu.SemaphoreType.DMA] * _NSET,
            [pltpu.SemaphoreType.DMA] * _NSET,
        ],
    )
    def gather_kernel(table_hbm, idx_hbm, out_hbm, idx_v, rows_v, gsems, wsems):
        wid = lax.axis_index("s") * _NC + lax.axis_index("c")
        base = wid * per_w
        pltpu.sync_copy(idx_hbm.at[wid], idx_v)

        def issue_gather(g, s):
            pltpu.async_copy(table_hbm.at[idx_v.at[g]], rows_v.at[s], gsems[s])

        def wait_gather(s):
            pltpu.make_async_copy(
                table_hbm.at[idx_v.at[0]], rows_v.at[s], gsems[s]
            ).wait()

        def issue_write(g, s):
            pltpu.async_copy(
                rows_v.at[s, pl.ds(0, _C), pl.ds(0, d)],
                out_hbm.at[pl.ds(base + g * _C, _C), pl.ds(0, d)],
                wsems[s],
            )

        def wait_write(s):
            pltpu.make_async_copy(
                rows_v.at[s, pl.ds(0, _C), pl.ds(0, d)],
                out_hbm.at[pl.ds(base, _C), pl.ds(0, d)],
                wsems[s],
            ).wait()

        for s in range(_LAG):
            issue_gather(s, s)

        def group(j, carry):
            for u in range(_NSET):
                sg = j * _NSET + u
                wait_gather(u)
                issue_write(sg, u)
                t = sg + _LAG
                s_t = (u + _LAG) % _NSET
                if u == 0:
                    @pl.when((t < n_sg) & (sg >= 1))
                    def _ww():
                        wait_write(s_t)
                else:
                    @pl.when(t < n_sg)
                    def _ww():
                        wait_write(s_t)

                @pl.when(t < n_sg)
                def _ig():
                    issue_gather(t, s_t)

            return carry

        lax.fori_loop(0, n_sg // _NSET, group, 0)
        for u in range(_NSET):
            wait_write(u)

    return gather_kernel


def kernel(x, table):
    b, l = x.shape
    v, d = table.shape
    n = b * l
    per_w = n // _NW
    table_p = jnp.pad(table, ((0, 0), (0, _W - d)))
    idx = x.reshape(_NW, per_w // _C, _C)
    out_p = _make_gather(n, v, d)(table_p, idx)
    return out_p[:, :d].reshape(b, l, d)
